# Initial kernel scaffold; baseline (speedup 1.0000x reference)
import jax
import jax.numpy as jnp
from jax.experimental import pallas as pl


def kernel(x, edge_index, edge_weight, batch, W, b_conv, gn_weight, gn_bias, gn_mean_scale):
    raise NotImplementedError("write your pallas kernel here")


# speedup vs baseline: 15.7676x; 15.7676x over previous
import functools

import jax
import jax.numpy as jnp
from jax import lax
from jax.experimental import pallas as pl
from jax.experimental.pallas import tpu as pltpu
from jax.experimental.pallas import tpu_sc as plsc

N = 10000
E = 320000
D = 128
G = 64

NC = 2
NS = 16
L = 16
NW = NC * NS
CHUNK = 128
ET = E // NW
NCH = -(-ET // CHUNK)
ETP = NCH * CHUNK
EP = NW * ETP
N_PAD = 10240
STRIPE = N_PAD // NS

_mesh = plsc.VectorSubcoreMesh(core_axis_name="c", subcore_axis_name="s")
_sc_params = pltpu.CompilerParams(needs_layout_passes=False)


@functools.partial(
    pl.kernel,
    out_type=jax.ShapeDtypeStruct((NW, N_PAD), jnp.float32),
    mesh=_mesh,
    compiler_params=_sc_params,
    scratch_types=[
        pltpu.VMEM((ETP,), jnp.int32),
        pltpu.VMEM((ETP,), jnp.float32),
        pltpu.VMEM((N_PAD,), jnp.float32),
    ],
)
def _deg_kernel(col_hbm, ew_hbm, deg_out, col_v, ew_v, deg_v):
    wid = lax.axis_index("s") * NC + lax.axis_index("c")
    pltpu.sync_copy(col_hbm.at[wid], col_v)
    pltpu.sync_copy(ew_hbm.at[wid], ew_v)
    zeros = jnp.zeros((L,), jnp.float32)

    def zbody(i, carry):
        deg_v[pl.ds(pl.multiple_of(i * L, L), L)] = zeros
        return carry

    lax.fori_loop(0, N_PAD // L, zbody, 0)

    def ebody(i, carry):
        off = pl.ds(pl.multiple_of(i * L, L), L)
        plsc.addupdate_scatter(deg_v, [col_v[off]], ew_v[off])
        return carry

    lax.fori_loop(0, ETP // L, ebody, 0)
    pltpu.sync_copy(deg_v, deg_out.at[wid])


def _prep_body(x_ref, w_ref, degp_ref, hp_ref, dis_ref):
    deg = jnp.sum(degp_ref[...], axis=0)[:N] + 1.0
    dis = jnp.where(deg > 0, lax.rsqrt(deg), 0.0)
    h = jnp.dot(x_ref[...], w_ref[...], preferred_element_type=jnp.float32)
    hp_ref[...] = h * dis[:, None]
    dis_ref[...] = dis


_prep_call = pl.pallas_call(
    _prep_body,
    out_shape=(
        jax.ShapeDtypeStruct((N, D), jnp.float32),
        jax.ShapeDtypeStruct((N,), jnp.float32),
    ),
)


@functools.partial(
    pl.kernel,
    out_type=jax.ShapeDtypeStruct((NC, N_PAD, D), jnp.float32),
    mesh=_mesh,
    compiler_params=_sc_params,
    scratch_types=[
        pltpu.VMEM((NCH, CHUNK), jnp.int32),
        pltpu.VMEM((NCH, CHUNK), jnp.int32),
        pltpu.VMEM((ETP,), jnp.float32),
        pltpu.VMEM((CHUNK, D), jnp.float32),
        pltpu.VMEM_SHARED((N_PAD, D), jnp.float32),
        pltpu.SemaphoreType.DMA,
    ],
)
def _msg_kernel(row_hbm, col_hbm, ew_hbm, hp_hbm, out_hbm,
                row_v, col_v, ew_v, buf, acc, sem):
    cid = lax.axis_index("c")
    sid = lax.axis_index("s")
    wid = sid * NC + cid
    pltpu.sync_copy(row_hbm.at[wid], row_v)
    pltpu.sync_copy(col_hbm.at[wid], col_v)
    pltpu.sync_copy(ew_hbm.at[wid], ew_v)

    zeros = jnp.zeros((L,), jnp.float32)

    def zbody(i, carry):
        r = i // (D // L)
        c = (i % (D // L)) * L
        buf[r, pl.ds(c, L)] = zeros
        return carry

    lax.fori_loop(0, CHUNK * D // L, zbody, 0)
    for k in range(STRIPE // CHUNK):
        pltpu.sync_copy(buf, acc.at[pl.ds(sid * STRIPE + k * CHUNK, CHUNK)])
    plsc.subcore_barrier()

    def chunk_body(ch, carry):
        pltpu.async_copy(hp_hbm.at[row_v.at[ch]], buf, sem).wait()
        base = pl.multiple_of(ch * CHUNK, CHUNK)
        for g in range(CHUNK // L):
            ew_g = ew_v[pl.ds(base + g * L, L)]
            for i in range(L):
                s = jnp.take_along_axis(
                    ew_g, jnp.full((L,), i, dtype=jnp.int32), axis=0)
                r = g * L + i
                for j in range(D // L):
                    sl = pl.ds(j * L, L)
                    buf[r, sl] = buf[r, sl] * s
        pltpu.sync_copy(buf, acc.at[col_v.at[ch]], add=True)
        return carry

    lax.fori_loop(0, NCH, chunk_body, 0)
    plsc.subcore_barrier()
    pltpu.sync_copy(
        acc.at[pl.ds(sid * STRIPE, STRIPE)],
        out_hbm.at[cid, pl.ds(sid * STRIPE, STRIPE)],
    )


def _post_body(sp_ref, hp_ref, dis_ref, bconv_ref, batch_ref,
               gnw_ref, gnb_ref, gms_ref, y_ref):
    s = sp_ref[0, :N, :] + sp_ref[1, :N, :]
    dis = dis_ref[...]
    out = dis[:, None] * (s + hp_ref[...]) + bconv_ref[...]

    batch = batch_ref[...]
    gids = lax.iota(jnp.int32, G)
    oh_ng = (batch[:, None] == gids[None, :]).astype(jnp.float32)
    cnt = jnp.maximum(jnp.sum(oh_ng, axis=0), 1.0)
    sums = lax.dot_general(oh_ng, out, (((0,), (0,)), ((), ())),
                           preferred_element_type=jnp.float32)
    sumsq = lax.dot_general(oh_ng, out * out, (((0,), (0,)), ((), ())),
                            preferred_element_type=jnp.float32)
    mean = sums / cnt[:, None]
    m2 = sumsq / cnt[:, None]
    gms = gms_ref[...]
    var = m2 + (gms * gms - 2.0 * gms) * (mean * mean)
    inv_std = lax.rsqrt(var + 1e-5)
    mean_row = jnp.dot(oh_ng, mean, preferred_element_type=jnp.float32)
    isd_row = jnp.dot(oh_ng, inv_std, preferred_element_type=jnp.float32)
    out_c = out - mean_row * gms
    y = gnw_ref[...] * out_c * isd_row + gnb_ref[...]
    y_ref[...] = jnp.where(y > 0, y, 0.1 * y)


_post_call = pl.pallas_call(
    _post_body,
    out_shape=jax.ShapeDtypeStruct((N, D), jnp.float32),
)


def kernel(x, edge_index, edge_weight, batch, W, b_conv, gn_weight, gn_bias,
           gn_mean_scale):
    row = edge_index[0].astype(jnp.int32)
    col = edge_index[1].astype(jnp.int32)
    batch32 = batch.astype(jnp.int32)
    pad = EP - E
    rowp = jnp.concatenate([row, jnp.zeros((pad,), jnp.int32)])
    colp = jnp.concatenate([col, jnp.zeros((pad,), jnp.int32)])
    ewp = jnp.concatenate([edge_weight.astype(jnp.float32),
                           jnp.zeros((pad,), jnp.float32)])
    row2 = rowp.reshape(NW, NCH, CHUNK)
    col2 = colp.reshape(NW, NCH, CHUNK)
    colf = colp.reshape(NW, ETP)
    ewf = ewp.reshape(NW, ETP)

    degp = _deg_kernel(colf, ewf)
    hp, dis = _prep_call(x, W, degp)
    spart = _msg_kernel(row2, col2, ewf, hp)
    y = _post_call(spart, hp, dis, b_conv, batch32, gn_weight, gn_bias,
                   gn_mean_scale)
    return y
